# Initial kernel scaffold; baseline (speedup 1.0000x reference)
import jax
import jax.numpy as jnp
from jax.experimental import pallas as pl


def kernel(x, edge_index, batch, W1, b1, bn1_w, bn1_b, W2, b2, bn2_w, bn2_b, lin_W, lin_b):
    raise NotImplementedError("write your pallas kernel here")


# speedup vs baseline: 30.4932x; 30.4932x over previous
import functools

import jax
import jax.numpy as jnp
from jax import lax
from jax.experimental import pallas as pl
from jax.experimental.pallas import tpu as pltpu
from jax.experimental.pallas import tpu_sc as plsc

N = 10000
NPAD = 10240
E = 320000
NC = 2
NS = 16
NW = NC * NS
CHUNK = 128
NCHUNK = 79
PER_TILE = NCHUNK * CHUNK
EPAD = PER_TILE * NW
ROWS_PER_TILE = NPAD // NS
EPS = 1e-5

_mesh = functools.partial(
    plsc.VectorSubcoreMesh, core_axis_name="c", subcore_axis_name="s")


def _zero_fill(buf, rows, cols):
  zero = jnp.zeros((16,), jnp.float32)
  cpr = cols // 16

  def body(i, _):
    buf[i // cpr, pl.ds((i % cpr) * 16, 16)] = zero
    return 0

  lax.fori_loop(0, rows * cpr, body, 0)


def _make_deg_kernel():
  C = 16

  @functools.partial(
      pl.kernel,
      mesh=_mesh(),
      out_type=jax.ShapeDtypeStruct((NC, NPAD, C), jnp.float32),
      compiler_params=pltpu.CompilerParams(use_tc_tiling_on_sc=False),
      scratch_types=[
          pltpu.VMEM((NCHUNK, CHUNK), jnp.int32),
          pltpu.VMEM((CHUNK, C), jnp.float32),
          pltpu.VMEM((CHUNK, C), jnp.float32),
          pltpu.VMEM_SHARED((NPAD, C), jnp.float32),
      ],
  )
  def deg_kernel(dst_hbm, out_hbm, dst_v, ones_v, zbuf, acc_sh):
    cid = lax.axis_index("c")
    sid = lax.axis_index("s")
    wid = cid * NS + sid

    _zero_fill(zbuf, CHUNK, C)
    one = jnp.full((16,), 1.0, jnp.float32)

    def fill_ones(i, _):
      ones_v[i, pl.ds(0, 16)] = one
      return 0

    lax.fori_loop(0, CHUNK, fill_ones, 0)

    def zseg(j, _):
      pltpu.sync_copy(zbuf, acc_sh.at[pl.ds(sid * ROWS_PER_TILE + j * CHUNK,
                                            CHUNK)])
      return 0

    lax.fori_loop(0, ROWS_PER_TILE // CHUNK, zseg, 0)
    pltpu.sync_copy(dst_hbm.at[wid], dst_v)
    plsc.subcore_barrier()

    def scat(j, _):
      pltpu.sync_copy(ones_v, acc_sh.at[dst_v.at[j]], add=True)
      return 0

    lax.fori_loop(0, NCHUNK, scat, 0)
    plsc.subcore_barrier()

    pltpu.sync_copy(
        acc_sh.at[pl.ds(sid * ROWS_PER_TILE, ROWS_PER_TILE)],
        out_hbm.at[cid, pl.ds(sid * ROWS_PER_TILE, ROWS_PER_TILE)])

  return deg_kernel


def _make_conv_kernel(C):
  @functools.partial(
      pl.kernel,
      mesh=_mesh(),
      out_type=jax.ShapeDtypeStruct((NC, NPAD, C), jnp.float32),
      compiler_params=pltpu.CompilerParams(use_tc_tiling_on_sc=False),
      scratch_types=[
          pltpu.VMEM((NCHUNK, CHUNK), jnp.int32),
          pltpu.VMEM((NCHUNK, CHUNK), jnp.int32),
          pltpu.VMEM((CHUNK, C), jnp.float32),
          pltpu.VMEM((CHUNK, C), jnp.float32),
          pltpu.VMEM_SHARED((NPAD, C), jnp.float32),
          pltpu.SemaphoreType.DMA,
      ],
  )
  def conv_kernel(y_hbm, src_hbm, dst_hbm, out_hbm,
                  src_v, dst_v, rows_v, zbuf, acc_sh, sem):
    cid = lax.axis_index("c")
    sid = lax.axis_index("s")
    wid = cid * NS + sid

    _zero_fill(zbuf, CHUNK, C)

    def zseg(j, _):
      pltpu.sync_copy(zbuf, acc_sh.at[pl.ds(sid * ROWS_PER_TILE + j * CHUNK,
                                            CHUNK)])
      return 0

    lax.fori_loop(0, ROWS_PER_TILE // CHUNK, zseg, 0)
    pltpu.sync_copy(src_hbm.at[wid], src_v)
    pltpu.sync_copy(dst_hbm.at[wid], dst_v)
    plsc.subcore_barrier()

    def edge_chunk(j, _):
      pltpu.async_copy(y_hbm.at[src_v.at[j]], rows_v, sem).wait()
      pltpu.sync_copy(rows_v, acc_sh.at[dst_v.at[j]], add=True)
      return 0

    lax.fori_loop(0, NCHUNK, edge_chunk, 0)
    plsc.subcore_barrier()

    pltpu.sync_copy(
        acc_sh.at[pl.ds(sid * ROWS_PER_TILE, ROWS_PER_TILE)],
        out_hbm.at[cid, pl.ds(sid * ROWS_PER_TILE, ROWS_PER_TILE)])

  return conv_kernel


_deg_kernel = _make_deg_kernel()
_conv16 = _make_conv_kernel(16)
_conv32 = _make_conv_kernel(32)


def _tc1_body(degp_ref, x_ref, w1_ref, dinv_ref, y1_ref):
  deg = degp_ref[0] + degp_ref[1] + 1.0
  dinv = lax.rsqrt(deg)
  xw = jnp.dot(x_ref[...], w1_ref[...], preferred_element_type=jnp.float32)
  dinv_ref[...] = dinv
  y1_ref[...] = xw * dinv


def _tc2_body(accp_ref, y1_ref, dinv_ref, w2_ref, cvec_ref, y2_ref):
  acc = accp_ref[0] + accp_ref[1] + y1_ref[...]
  conv = acc * dinv_ref[...] + cvec_ref[0:1, :]
  h = jnp.maximum(conv * cvec_ref[1:2, :] + cvec_ref[2:3, :], 0.0)
  h2 = jnp.dot(h, w2_ref[...], preferred_element_type=jnp.float32)
  y2_ref[...] = h2 * dinv_ref[:, 0:1]


def _tc3_body(accp_ref, y2_ref, dinv_ref, batch_ref, cvec_ref,
              linw_ref, linb_ref, out_ref):
  acc = accp_ref[0] + accp_ref[1] + y2_ref[...]
  conv = acc * dinv_ref[:, 0:1] + cvec_ref[0:1, :]
  h = jnp.maximum(conv * cvec_ref[1:2, :] + cvec_ref[2:3, :], 0.0)
  ones_col = jnp.ones((NPAD, 1), jnp.float32)
  he = jnp.concatenate([h, ones_col], axis=1)
  gids = lax.broadcasted_iota(jnp.int32, (64, NPAD), 0)
  p = (batch_ref[...] == gids).astype(jnp.float32)
  se = jnp.dot(p, he, preferred_element_type=jnp.float32)
  pooled = se[:, :32] / jnp.maximum(se[:, 32:33], 1.0)
  out_ref[...] = jnp.dot(pooled, linw_ref[...],
                         preferred_element_type=jnp.float32) + linb_ref[...]


def kernel(x, edge_index, batch, W1, b1, bn1_w, bn1_b, W2, b2, bn2_w, bn2_b,
           lin_W, lin_b):
  f32 = jnp.float32
  src = edge_index[0].astype(jnp.int32)
  dst = edge_index[1].astype(jnp.int32)
  pad = EPAD - E
  src_p = jnp.concatenate([src, jnp.zeros((pad,), jnp.int32)])
  src_p = src_p.reshape(NW, NCHUNK, CHUNK)
  dst_p = jnp.concatenate([dst, jnp.full((pad,), N, jnp.int32)])
  dst_p = dst_p.reshape(NW, NCHUNK, CHUNK)
  x_p = jnp.concatenate([x, jnp.zeros((NPAD - N, x.shape[1]), f32)])
  batch_p = jnp.concatenate(
      [batch.astype(jnp.int32), jnp.full((NPAD - N,), 64, jnp.int32)])
  batch_p = batch_p.reshape(1, NPAD)

  bn_scale1 = bn1_w * (1.0 / jnp.sqrt(1.0 + EPS))
  bn_scale2 = bn2_w * (1.0 / jnp.sqrt(1.0 + EPS))
  cvec1 = jnp.stack([b1, bn_scale1, bn1_b])
  cvec2 = jnp.stack([b2, bn_scale2, bn2_b])

  degp = _deg_kernel(dst_p)

  dinv, y1 = pl.pallas_call(
      _tc1_body,
      out_shape=(jax.ShapeDtypeStruct((NPAD, 16), f32),
                 jax.ShapeDtypeStruct((NPAD, 16), f32)),
  )(degp, x_p, W1)

  acc1 = _conv16(y1, src_p, dst_p)

  y2 = pl.pallas_call(
      _tc2_body,
      out_shape=jax.ShapeDtypeStruct((NPAD, 32), f32),
  )(acc1, y1, dinv, W2, cvec1)

  acc2 = _conv32(y2, src_p, dst_p)

  out = pl.pallas_call(
      _tc3_body,
      out_shape=jax.ShapeDtypeStruct((64, 64), f32),
  )(acc2, y2, dinv, batch_p, cvec2, lin_W, lin_b.reshape(1, 64))

  return out


# speedup vs baseline: 39.7359x; 1.3031x over previous
import functools

import jax
import jax.numpy as jnp
from jax import lax
from jax.experimental import pallas as pl
from jax.experimental.pallas import tpu as pltpu
from jax.experimental.pallas import tpu_sc as plsc

N = 10000
NPAD = 10240
E = 320000
NC = 2
NS = 16
NW = NC * NS
CHUNK = 128
NCHUNK = 79
PER_TILE = NCHUNK * CHUNK
EPAD = PER_TILE * NW
ROWS_PER_TILE = NPAD // NS
EPS = 1e-5

_mesh = functools.partial(
    plsc.VectorSubcoreMesh, core_axis_name="c", subcore_axis_name="s")


def _zero_fill(buf, rows, cols):
  zero = jnp.zeros((16,), jnp.float32)
  cpr = cols // 16

  def body(i, _):
    buf[i // cpr, pl.ds((i % cpr) * 16, 16)] = zero
    return 0

  lax.fori_loop(0, rows * cpr, body, 0)


def _make_deg_kernel():
  C = 16

  @functools.partial(
      pl.kernel,
      mesh=_mesh(),
      out_type=jax.ShapeDtypeStruct((NC, NPAD, C), jnp.float32),
      compiler_params=pltpu.CompilerParams(use_tc_tiling_on_sc=False),
      scratch_types=[
          pltpu.VMEM((NCHUNK, CHUNK), jnp.int32),
          pltpu.VMEM((CHUNK, C), jnp.float32),
          pltpu.VMEM((CHUNK, C), jnp.float32),
          pltpu.VMEM_SHARED((NPAD, C), jnp.float32),
      ],
  )
  def deg_kernel(dst_hbm, out_hbm, dst_v, ones_v, zbuf, acc_sh):
    cid = lax.axis_index("c")
    sid = lax.axis_index("s")
    wid = cid * NS + sid

    _zero_fill(zbuf, CHUNK, C)
    one = jnp.full((16,), 1.0, jnp.float32)

    def fill_ones(i, _):
      ones_v[i, pl.ds(0, 16)] = one
      return 0

    lax.fori_loop(0, CHUNK, fill_ones, 0)

    def zseg(j, _):
      pltpu.sync_copy(zbuf, acc_sh.at[pl.ds(sid * ROWS_PER_TILE + j * CHUNK,
                                            CHUNK)])
      return 0

    lax.fori_loop(0, ROWS_PER_TILE // CHUNK, zseg, 0)
    pltpu.sync_copy(dst_hbm.at[wid], dst_v)
    plsc.subcore_barrier()

    def scat(j, _):
      pltpu.sync_copy(ones_v, acc_sh.at[dst_v.at[j]], add=True)
      return 0

    lax.fori_loop(0, NCHUNK, scat, 0)
    plsc.subcore_barrier()

    pltpu.sync_copy(
        acc_sh.at[pl.ds(sid * ROWS_PER_TILE, ROWS_PER_TILE)],
        out_hbm.at[cid, pl.ds(sid * ROWS_PER_TILE, ROWS_PER_TILE)])

  return deg_kernel


def _make_conv_kernel(C):
  @functools.partial(
      pl.kernel,
      mesh=_mesh(),
      out_type=jax.ShapeDtypeStruct((NC, NPAD, C), jnp.float32),
      compiler_params=pltpu.CompilerParams(use_tc_tiling_on_sc=False),
      scratch_types=[
          pltpu.VMEM((NCHUNK, CHUNK), jnp.int32),
          pltpu.VMEM((NCHUNK, CHUNK), jnp.int32),
          pltpu.VMEM((2, CHUNK, C), jnp.float32),
          pltpu.VMEM((CHUNK, C), jnp.float32),
          pltpu.VMEM_SHARED((NPAD, C), jnp.float32),
          pltpu.SemaphoreType.DMA,
          pltpu.SemaphoreType.DMA,
      ],
  )
  def conv_kernel(y_hbm, src_hbm, dst_hbm, out_hbm,
                  src_v, dst_v, rows_v, zbuf, acc_sh, sem0, sem1):
    cid = lax.axis_index("c")
    sid = lax.axis_index("s")
    wid = cid * NS + sid

    _zero_fill(zbuf, CHUNK, C)

    def zseg(j, _):
      pltpu.sync_copy(zbuf, acc_sh.at[pl.ds(sid * ROWS_PER_TILE + j * CHUNK,
                                            CHUNK)])
      return 0

    lax.fori_loop(0, ROWS_PER_TILE // CHUNK, zseg, 0)
    pltpu.sync_copy(src_hbm.at[wid], src_v)
    pltpu.sync_copy(dst_hbm.at[wid], dst_v)
    plsc.subcore_barrier()

    sems = (sem0, sem1)

    def issue(j, b):
      pltpu.async_copy(y_hbm.at[src_v.at[j]], rows_v.at[b], sems[b])

    def wait_scatter(j, b):
      pltpu.make_async_copy(y_hbm.at[src_v.at[j]], rows_v.at[b],
                            sems[b]).wait()
      pltpu.sync_copy(rows_v.at[b], acc_sh.at[dst_v.at[j]], add=True)

    issue(0, 0)

    def edge_chunk(j, _):
      nxt = j + 1
      more = nxt < NCHUNK
      odd_j = (j % 2) == 1

      @pl.when(jnp.logical_and(more, odd_j))
      def _():
        issue(nxt, 0)

      @pl.when(jnp.logical_and(more, jnp.logical_not(odd_j)))
      def _():
        issue(nxt, 1)

      @pl.when(odd_j)
      def _():
        wait_scatter(j, 1)

      @pl.when(jnp.logical_not(odd_j))
      def _():
        wait_scatter(j, 0)

      return 0

    lax.fori_loop(0, NCHUNK, edge_chunk, 0)
    plsc.subcore_barrier()

    pltpu.sync_copy(
        acc_sh.at[pl.ds(sid * ROWS_PER_TILE, ROWS_PER_TILE)],
        out_hbm.at[cid, pl.ds(sid * ROWS_PER_TILE, ROWS_PER_TILE)])

  return conv_kernel


_deg_kernel = _make_deg_kernel()
_conv16 = _make_conv_kernel(16)
_conv32 = _make_conv_kernel(32)


def _tc1_body(degp_ref, x_ref, w1_ref, dinv_ref, y1_ref):
  deg = degp_ref[0] + degp_ref[1] + 1.0
  dinv = lax.rsqrt(deg)
  xw = jnp.dot(x_ref[...], w1_ref[...], preferred_element_type=jnp.float32)
  dinv_ref[...] = dinv
  y1_ref[...] = xw * dinv


def _tc2_body(accp_ref, y1_ref, dinv_ref, w2_ref, cvec_ref, y2_ref):
  acc = accp_ref[0] + accp_ref[1] + y1_ref[...]
  conv = acc * dinv_ref[...] + cvec_ref[0:1, :]
  h = jnp.maximum(conv * cvec_ref[1:2, :] + cvec_ref[2:3, :], 0.0)
  h2 = jnp.dot(h, w2_ref[...], preferred_element_type=jnp.float32)
  y2_ref[...] = h2 * dinv_ref[:, 0:1]


def _tc3_body(accp_ref, y2_ref, dinv_ref, batch_ref, cvec_ref,
              linw_ref, linb_ref, out_ref):
  acc = accp_ref[0] + accp_ref[1] + y2_ref[...]
  conv = acc * dinv_ref[:, 0:1] + cvec_ref[0:1, :]
  h = jnp.maximum(conv * cvec_ref[1:2, :] + cvec_ref[2:3, :], 0.0)
  ones_col = jnp.ones((NPAD, 1), jnp.float32)
  he = jnp.concatenate([h, ones_col], axis=1)
  gids = lax.broadcasted_iota(jnp.int32, (64, NPAD), 0)
  p = (batch_ref[...] == gids).astype(jnp.float32)
  se = jnp.dot(p, he, preferred_element_type=jnp.float32)
  pooled = se[:, :32] / jnp.maximum(se[:, 32:33], 1.0)
  out_ref[...] = jnp.dot(pooled, linw_ref[...],
                         preferred_element_type=jnp.float32) + linb_ref[...]


def kernel(x, edge_index, batch, W1, b1, bn1_w, bn1_b, W2, b2, bn2_w, bn2_b,
           lin_W, lin_b):
  f32 = jnp.float32
  src = edge_index[0].astype(jnp.int32)
  dst = edge_index[1].astype(jnp.int32)
  pad = EPAD - E
  src_p = jnp.concatenate([src, jnp.zeros((pad,), jnp.int32)])
  src_p = src_p.reshape(NW, NCHUNK, CHUNK)
  dst_p = jnp.concatenate([dst, jnp.full((pad,), N, jnp.int32)])
  dst_p = dst_p.reshape(NW, NCHUNK, CHUNK)
  x_p = jnp.concatenate([x, jnp.zeros((NPAD - N, x.shape[1]), f32)])
  batch_p = jnp.concatenate(
      [batch.astype(jnp.int32), jnp.full((NPAD - N,), 64, jnp.int32)])
  batch_p = batch_p.reshape(1, NPAD)

  bn_scale1 = bn1_w * (1.0 / jnp.sqrt(1.0 + EPS))
  bn_scale2 = bn2_w * (1.0 / jnp.sqrt(1.0 + EPS))
  cvec1 = jnp.stack([b1, bn_scale1, bn1_b])
  cvec2 = jnp.stack([b2, bn_scale2, bn2_b])

  degp = _deg_kernel(dst_p)

  dinv, y1 = pl.pallas_call(
      _tc1_body,
      out_shape=(jax.ShapeDtypeStruct((NPAD, 16), f32),
                 jax.ShapeDtypeStruct((NPAD, 16), f32)),
  )(degp, x_p, W1)

  acc1 = _conv16(y1, src_p, dst_p)

  y2 = pl.pallas_call(
      _tc2_body,
      out_shape=jax.ShapeDtypeStruct((NPAD, 32), f32),
  )(acc1, y1, dinv, W2, cvec1)

  acc2 = _conv32(y2, src_p, dst_p)

  out = pl.pallas_call(
      _tc3_body,
      out_shape=jax.ShapeDtypeStruct((64, 64), f32),
  )(acc2, y2, dinv, batch_p, cvec2, lin_W, lin_b.reshape(1, 64))

  return out
